# Initial kernel scaffold; baseline (speedup 1.0000x reference)
import jax
import jax.numpy as jnp
from jax.experimental import pallas as pl


def kernel(x, edge_index, W):
    raise NotImplementedError("write your pallas kernel here")


# speedup vs baseline: 7.3498x; 7.3498x over previous
import functools

import jax
import jax.numpy as jnp
from jax import lax
from jax.experimental import pallas as pl
from jax.experimental.pallas import tpu as pltpu
from jax.experimental.pallas import tpu_sc as plsc

N = 10000
D = 128
E = 320000
NC = 2
NS = 16
NW = NC * NS
EPW = E // NW
CHUNK = 80
IDXC = 5
GROUPS = EPW // (IDXC * CHUNK)
RPT = 624
LAST_RPT = N - 15 * RPT
LANES = 16


def _edge_kernel_body(x_hbm, src_hbm, dst_hbm, out_hbm,
                      acc, src_idx, dst_idx, sbuf, dbuf, mbuf,
                      sem_s, sem_d):
    cid = lax.axis_index("c")
    sid = lax.axis_index("s")
    wid = sid * NC + cid

    zero16 = jnp.zeros((LANES,), jnp.float32)

    def zrow(r, carry):
        for j in range(D // LANES):
            mbuf[r, pl.ds(LANES * j, LANES)] = zero16
        return carry

    lax.fori_loop(0, CHUNK, zrow, 0)
    base = sid * RPT
    for j in range(RPT // CHUNK):
        pltpu.sync_copy(mbuf, acc.at[pl.ds(base + j * CHUNK, CHUNK)])

    @pl.when(sid == NS - 1)
    def _zero_tail_full():
        pltpu.sync_copy(mbuf, acc.at[pl.ds(base + 560, CHUNK)])

    @pl.when(sid < NS - 1)
    def _zero_tail_part():
        pltpu.sync_copy(mbuf.at[pl.ds(0, 64)], acc.at[pl.ds(base + 560, 64)])

    plsc.subcore_barrier()

    def lane_sum(t):
        r = t + lax.rev(t, (0,))
        return (((r[0] + r[1]) + (r[2] + r[3]))
                + ((r[4] + r[5]) + (r[6] + r[7])))

    def group_body(g, carry):
        pltpu.sync_copy(src_hbm.at[wid, g], src_idx)
        pltpu.sync_copy(dst_hbm.at[wid, g], dst_idx)

        def chunk_body(c, carry2):
            gs = pltpu.async_copy(x_hbm.at[src_idx.at[c]], sbuf, sem_s)
            gd = pltpu.async_copy(x_hbm.at[dst_idx.at[c]], dbuf, sem_d)
            gs.wait()
            gd.wait()

            @plsc.parallel_loop(0, CHUNK, unroll=2)
            def edge_body(e):
                s = [sbuf[e, pl.ds(LANES * j, LANES)]
                     for j in range(D // LANES)]
                d = [dbuf[e, pl.ds(LANES * j, LANES)]
                     for j in range(D // LANES)]
                p = [s[j] * d[j] for j in range(D // LANES)]
                t0 = (p[0] + p[1]) + (p[2] + p[3])
                t1 = (p[4] + p[5]) + (p[6] + p[7])
                a = lane_sum(t0 + t1)
                for j in range(D // LANES):
                    mbuf[e, pl.ds(LANES * j, LANES)] = s[j] * a

            pltpu.sync_copy(mbuf, acc.at[dst_idx.at[c]], add=True)
            return carry2

        lax.fori_loop(0, IDXC, chunk_body, 0)
        return carry

    lax.fori_loop(0, GROUPS, group_body, 0)

    plsc.subcore_barrier()
    pltpu.sync_copy(acc.at[pl.ds(base, RPT)],
                    out_hbm.at[cid, pl.ds(base, RPT)])

    @pl.when(sid == NS - 1)
    def _pub_tail():
        pltpu.sync_copy(acc.at[pl.ds(base + RPT, LAST_RPT - RPT)],
                        out_hbm.at[cid, pl.ds(base + RPT, LAST_RPT - RPT)])


_edge_kernel = functools.partial(
    pl.kernel,
    out_type=jax.ShapeDtypeStruct((NC, N, D), jnp.float32),
    mesh=plsc.VectorSubcoreMesh(core_axis_name="c", subcore_axis_name="s"),
    scratch_types=[
        pltpu.VMEM_SHARED((N, D), jnp.float32),
        pltpu.VMEM((IDXC, CHUNK), jnp.int32),
        pltpu.VMEM((IDXC, CHUNK), jnp.int32),
        pltpu.VMEM((CHUNK, D), jnp.float32),
        pltpu.VMEM((CHUNK, D), jnp.float32),
        pltpu.VMEM((CHUNK, D), jnp.float32),
        pltpu.SemaphoreType.DMA,
        pltpu.SemaphoreType.DMA,
    ],
)(_edge_kernel_body)


def _mm_body(p_ref, w_ref, o_ref):
    s = p_ref[0] + p_ref[1]
    o_ref[...] = lax.dot_general(
        s, w_ref[...], (((1,), (1,)), ((), ())),
        preferred_element_type=jnp.float32)


_MM_BLK = 2000


def _final_matmul(partial, W):
    return pl.pallas_call(
        _mm_body,
        grid=(N // _MM_BLK,),
        in_specs=[
            pl.BlockSpec((NC, _MM_BLK, D), lambda i: (0, i, 0)),
            pl.BlockSpec((D, D), lambda i: (0, 0)),
        ],
        out_specs=pl.BlockSpec((_MM_BLK, D), lambda i: (i, 0)),
        out_shape=jax.ShapeDtypeStruct((N, D), jnp.float32),
    )(partial, W)


def kernel(x, edge_index, W):
    src = edge_index[0].reshape(NW, GROUPS, IDXC, CHUNK)
    dst = edge_index[1].reshape(NW, GROUPS, IDXC, CHUNK)
    partial = _edge_kernel(x, src, dst)
    return _final_matmul(partial, W)


# speedup vs baseline: 8.8648x; 1.2061x over previous
import functools

import jax
import jax.numpy as jnp
from jax import lax
from jax.experimental import pallas as pl
from jax.experimental.pallas import tpu as pltpu
from jax.experimental.pallas import tpu_sc as plsc

N = 10000
D = 128
E = 320000
NC = 2
NS = 16
NW = NC * NS
EPW = E // NW
CHUNK = 25
GSIZE = 8
NGROUP = EPW // (GSIZE * CHUNK)
NSTEP = NGROUP // 2
RPT = 624
LAST_RPT = N - 15 * RPT
LANES = 16


def _edge_kernel_body(x_hbm, src_hbm, dst_hbm, out_hbm,
                      acc, sidx0, sidx1, didx0, didx1,
                      sbuf0, sbuf1, dbuf0, dbuf1, mbuf0, mbuf1,
                      sem_is, sem_id, sem_s0, sem_s1, sem_d0, sem_d1,
                      sem_m0, sem_m1):
    cid = lax.axis_index("c")
    sid = lax.axis_index("s")
    wid = sid * NC + cid

    sidx = (sidx0, sidx1)
    didx = (didx0, didx1)
    sbuf = (sbuf0, sbuf1)
    dbuf = (dbuf0, dbuf1)
    mbuf = (mbuf0, mbuf1)
    sem_s = (sem_s0, sem_s1)
    sem_d = (sem_d0, sem_d1)
    sem_m = (sem_m0, sem_m1)

    zero16 = jnp.zeros((LANES,), jnp.float32)

    def zrow(r, carry):
        for j in range(D // LANES):
            mbuf0[r, pl.ds(LANES * j, LANES)] = zero16
        return carry

    lax.fori_loop(0, CHUNK, zrow, 0)
    base = sid * RPT
    for j in range(RPT // 24):
        pltpu.sync_copy(mbuf0.at[pl.ds(0, 24)],
                        acc.at[pl.ds(base + j * 24, 24)])

    @pl.when(sid == NS - 1)
    def _zero_tail():
        pltpu.sync_copy(mbuf0.at[pl.ds(0, 16)],
                        acc.at[pl.ds(base + RPT, 16)])

    plsc.subcore_barrier()

    def lane_sum(t):
        r = t + lax.rev(t, (0,))
        return (((r[0] + r[1]) + (r[2] + r[3]))
                + ((r[4] + r[5]) + (r[6] + r[7])))

    def compute_chunk(par):
        @plsc.parallel_loop(0, CHUNK, unroll=2)
        def edge_body(e):
            s = [sbuf[par][e, pl.ds(LANES * j, LANES)]
                 for j in range(D // LANES)]
            d = [dbuf[par][e, pl.ds(LANES * j, LANES)]
                 for j in range(D // LANES)]
            p = [s[j] * d[j] for j in range(D // LANES)]
            t0 = (p[0] + p[1]) + (p[2] + p[3])
            t1 = (p[4] + p[5]) + (p[6] + p[7])
            a = lane_sum(t0 + t1)
            for j in range(D // LANES):
                mbuf[par][e, pl.ds(LANES * j, LANES)] = s[j] * a

    def start_gathers(gp, q):
        pltpu.async_copy(x_hbm.at[sidx[gp].at[q]], sbuf[q % 2],
                         sem_s[q % 2])
        pltpu.async_copy(x_hbm.at[didx[gp].at[q]], dbuf[q % 2],
                         sem_d[q % 2])

    def wait_gathers(par):
        pltpu.make_async_copy(x_hbm.at[sidx[0].at[0]], sbuf[par],
                              sem_s[par]).wait()
        pltpu.make_async_copy(x_hbm.at[sidx[0].at[0]], dbuf[par],
                              sem_d[par]).wait()

    def drain_scatter(par):
        pltpu.make_async_copy(mbuf[par], acc.at[sidx[0].at[0]],
                              sem_m[par]).wait()

    def handle_group(g, gp, first_group, last_group):
        for q in range(GSIZE):
            par = q % 2
            if q < 2:
                @pl.when(jnp.logical_not(first_group))
                def _drain_cond():
                    drain_scatter(par)
            else:
                drain_scatter(par)
            if q == 2:
                @pl.when(jnp.logical_not(last_group))
                def _prefetch_idx():
                    pltpu.async_copy(src_hbm.at[wid, g + 1], sidx[1 - gp],
                                     sem_is)
                    pltpu.async_copy(dst_hbm.at[wid, g + 1], didx[1 - gp],
                                     sem_id)
            if q == GSIZE - 2:
                @pl.when(jnp.logical_not(last_group))
                def _wait_idx():
                    pltpu.make_async_copy(src_hbm.at[wid, 0], sidx[1 - gp],
                                          sem_is).wait()
                    pltpu.make_async_copy(dst_hbm.at[wid, 0], didx[1 - gp],
                                          sem_id).wait()
            wait_gathers(par)
            compute_chunk(par)
            pltpu.async_copy(mbuf[par], acc.at[didx[gp].at[q]], sem_m[par],
                             add=True)
            if q < GSIZE - 2:
                start_gathers(gp, q + 2)
            else:
                @pl.when(jnp.logical_not(last_group))
                def _gather_next():
                    start_gathers(1 - gp, q + 2 - GSIZE)

    pltpu.sync_copy(src_hbm.at[wid, 0], sidx[0])
    pltpu.sync_copy(dst_hbm.at[wid, 0], didx[0])
    start_gathers(0, 0)
    start_gathers(0, 1)

    false_ = jnp.bool_(False)

    def step_body(i, carry):
        handle_group(2 * i, 0, i == 0, false_)
        handle_group(2 * i + 1, 1, false_, i == NSTEP - 1)
        return carry

    lax.fori_loop(0, NSTEP, step_body, 0)

    drain_scatter(0)
    drain_scatter(1)
    plsc.subcore_barrier()
    pltpu.sync_copy(acc.at[pl.ds(base, RPT)],
                    out_hbm.at[cid, pl.ds(base, RPT)])

    @pl.when(sid == NS - 1)
    def _pub_tail():
        pltpu.sync_copy(acc.at[pl.ds(base + RPT, LAST_RPT - RPT)],
                        out_hbm.at[cid, pl.ds(base + RPT, LAST_RPT - RPT)])


_edge_kernel = functools.partial(
    pl.kernel,
    out_type=jax.ShapeDtypeStruct((NC, N, D), jnp.float32),
    mesh=plsc.VectorSubcoreMesh(core_axis_name="c", subcore_axis_name="s"),
    scratch_types=[
        pltpu.VMEM_SHARED((N, D), jnp.float32),
        pltpu.VMEM((GSIZE, CHUNK), jnp.int32),
        pltpu.VMEM((GSIZE, CHUNK), jnp.int32),
        pltpu.VMEM((GSIZE, CHUNK), jnp.int32),
        pltpu.VMEM((GSIZE, CHUNK), jnp.int32),
        pltpu.VMEM((CHUNK, D), jnp.float32),
        pltpu.VMEM((CHUNK, D), jnp.float32),
        pltpu.VMEM((CHUNK, D), jnp.float32),
        pltpu.VMEM((CHUNK, D), jnp.float32),
        pltpu.VMEM((CHUNK, D), jnp.float32),
        pltpu.VMEM((CHUNK, D), jnp.float32),
        pltpu.SemaphoreType.DMA,
        pltpu.SemaphoreType.DMA,
        pltpu.SemaphoreType.DMA,
        pltpu.SemaphoreType.DMA,
        pltpu.SemaphoreType.DMA,
        pltpu.SemaphoreType.DMA,
        pltpu.SemaphoreType.DMA,
        pltpu.SemaphoreType.DMA,
    ],
)(_edge_kernel_body)


def _mm_body(p_ref, w_ref, o_ref):
    s = p_ref[0] + p_ref[1]
    o_ref[...] = lax.dot_general(
        s, w_ref[...], (((1,), (1,)), ((), ())),
        preferred_element_type=jnp.float32)


_MM_BLK = 2000


def _final_matmul(partial, W):
    return pl.pallas_call(
        _mm_body,
        grid=(N // _MM_BLK,),
        in_specs=[
            pl.BlockSpec((NC, _MM_BLK, D), lambda i: (0, i, 0)),
            pl.BlockSpec((D, D), lambda i: (0, 0)),
        ],
        out_specs=pl.BlockSpec((_MM_BLK, D), lambda i: (i, 0)),
        out_shape=jax.ShapeDtypeStruct((N, D), jnp.float32),
    )(partial, W)


def kernel(x, edge_index, W):
    src = edge_index[0].reshape(NW, NGROUP, GSIZE, CHUNK)
    dst = edge_index[1].reshape(NW, NGROUP, GSIZE, CHUNK)
    partial = _edge_kernel(x, src, dst)
    return _final_matmul(partial, W)
